# baseline (speedup 1.0000x reference)
import functools

import jax
import jax.numpy as jnp
from jax import lax
from jax.experimental import pallas as pl
from jax.experimental.pallas import tpu as pltpu
from jax.experimental.pallas import tpu_sc as plsc

NC = 2
NS = 16
NW = NC * NS
LANE = 128
SUB = 8


@functools.lru_cache(maxsize=None)
def _build(batch, seq, dim):
    bt = batch // LANE
    assert bt == NW and seq % SUB == 0 and dim % 16 == 0
    sg = seq // SUB
    jg = dim // SUB
    lg = LANE // 16

    mesh = plsc.VectorSubcoreMesh(core_axis_name="c", subcore_axis_name="s")

    @functools.partial(
        pl.kernel,
        mesh=mesh,
        out_type=jax.ShapeDtypeStruct((seq * SUB, bt, SUB, LANE), jnp.float32),
        compiler_params=pltpu.CompilerParams(
            use_tc_tiling_on_sc=False, needs_layout_passes=False
        ),
        scratch_types=[
            pltpu.VMEM((sg, SUB, LANE), jnp.int32),
            pltpu.VMEM((LANE, dim), jnp.float32),
            pltpu.VMEM((LANE, dim), jnp.float32),
            pltpu.VMEM((jg, SUB, LANE), jnp.float32),
            pltpu.SemaphoreType.DMA,
            pltpu.SemaphoreType.DMA,
        ],
    )
    def emb(ids_hbm, table_hbm, out_hbm, idx_v, buf0, buf1, buft, sem0, sem1):
        w = lax.axis_index("s") * NC + lax.axis_index("c")
        bufs = (buf0, buf1)
        sems = (sem0, sem1)

        pltpu.sync_copy(ids_hbm.at[:, w], idx_v)

        def fire(g, r, buf, sem):
            pltpu.make_async_copy(
                table_hbm.at[idx_v.at[g, r]], buf, sem
            ).start()

        def wait(buf, sem):
            pltpu.make_async_copy(table_hbm.at[idx_v.at[0, 0]], buf, sem).wait()

        rows = [jnp.arange(16, dtype=jnp.int32) + 16 * l for l in range(lg)]

        def transpose_write(g, r, buf):
            def tbody(a, carry):
                for b in range(SUB):
                    col = jnp.full((16,), a * SUB + b, dtype=jnp.int32)
                    for l in range(lg):
                        v = plsc.load_gather(buf, [rows[l], col])
                        buft[a, b, pl.ds(16 * l, 16)] = v
                return carry

            lax.fori_loop(0, jg, tbody, 0)
            s = g * SUB + r
            pltpu.sync_copy(buft, out_hbm.at[pl.ds(s * SUB, SUB), w])

        fire(0, 0, bufs[0], sems[0])

        def body(g, carry):
            for r in range(SUB):
                p = r & 1
                q = p ^ 1
                if r < SUB - 1:
                    fire(g, r + 1, bufs[q], sems[q])
                else:
                    @pl.when(g < sg - 1)
                    def _():
                        fire(g + 1, 0, bufs[q], sems[q])
                wait(bufs[p], sems[p])
                transpose_write(g, r, bufs[p])
            return carry

        lax.fori_loop(0, sg, body, 0)

    return emb


def kernel(word_ids, word_emb_table):
    batch, seq = word_ids.shape
    vocab, dim = word_emb_table.shape
    ids_lin = (
        word_ids.astype(jnp.int32)
        .T.reshape(seq // SUB, SUB, batch // LANE, LANE)
        .transpose(0, 2, 1, 3)
    )
    emb = _build(batch, seq, dim)
    out5 = emb(ids_lin, word_emb_table).reshape(
        seq, dim // SUB, batch // LANE, SUB, LANE
    )
    return out5.transpose(2, 4, 0, 1, 3).reshape(batch, seq, dim)


# speedup vs baseline: 1.0057x; 1.0057x over previous
import functools

import jax
import jax.numpy as jnp
from jax import lax
from jax.experimental import pallas as pl
from jax.experimental.pallas import tpu as pltpu
from jax.experimental.pallas import tpu_sc as plsc

NC = 2
NS = 16
NW = NC * NS
LANE = 128
SUB = 8
NBUF = 4


@functools.lru_cache(maxsize=None)
def _build(batch, seq, dim):
    bt = batch // LANE
    assert bt == NW and seq % SUB == 0 and dim % 16 == 0
    sg = seq // SUB
    jg = dim // SUB
    lg = LANE // 16

    mesh = plsc.VectorSubcoreMesh(core_axis_name="c", subcore_axis_name="s")

    @functools.partial(
        pl.kernel,
        mesh=mesh,
        out_type=jax.ShapeDtypeStruct((seq * SUB, bt, SUB, LANE), jnp.float32),
        compiler_params=pltpu.CompilerParams(
            use_tc_tiling_on_sc=False, needs_layout_passes=False
        ),
        scratch_types=[
            pltpu.VMEM((sg, SUB, LANE), jnp.int32),
            pltpu.VMEM((NBUF, LANE), jnp.int32),
            pltpu.VMEM((NBUF, LANE, 2 * dim), jnp.float32),
            pltpu.VMEM((2, jg, SUB, LANE), jnp.float32),
            [pltpu.SemaphoreType.DMA] * NBUF,
            [pltpu.SemaphoreType.DMA] * 2,
        ],
    )
    def emb(ids_hbm, table_hbm, out_hbm, idx_v, stage, bufs, buft, gsem, wsem):
        w = lax.axis_index("s") * NC + lax.axis_index("c")

        pltpu.sync_copy(ids_hbm.at[:, w], idx_v)

        def stage_and_fire(g, r, slot):
            for l in range(lg):
                iv = idx_v[g, r, pl.ds(16 * l, 16)]
                stage[slot, pl.ds(16 * l, 16)] = jax.lax.shift_right_logical(
                    iv, 1
                )
            pltpu.make_async_copy(
                table_hbm.at[stage.at[slot]], bufs.at[slot], gsem[slot]
            ).start()

        def wait_gather(slot):
            pltpu.make_async_copy(
                table_hbm.at[stage.at[slot]], bufs.at[slot], gsem[slot]
            ).wait()

        def wait_write(half):
            pltpu.make_async_copy(
                buft.at[half], out_hbm.at[pl.ds(0, SUB), w], wsem[half]
            ).wait()

        rows = [jnp.arange(16, dtype=jnp.int32) + 16 * l for l in range(lg)]

        def transpose_write(g, r, slot, half):
            par = []
            for l in range(lg):
                iv = idx_v[g, r, pl.ds(16 * l, 16)]
                par.append(jax.lax.shift_left(iv & 1, 6))

            def tbody(a, carry):
                for b in range(SUB):
                    col0 = jnp.full((16,), a * SUB + b, dtype=jnp.int32)
                    for l in range(lg):
                        v = plsc.load_gather(
                            bufs.at[slot], [rows[l], par[l] + col0]
                        )
                        buft[half, a, b, pl.ds(16 * l, 16)] = v
                return carry

            lax.fori_loop(0, jg, tbody, 0)
            s = g * SUB + r
            pltpu.make_async_copy(
                buft.at[half], out_hbm.at[pl.ds(s * SUB, SUB), w], wsem[half]
            ).start()

        for k in range(NBUF - 1):
            stage_and_fire(0, k, k)

        def body(g, carry):
            for r in range(SUB):
                slot = r % NBUF
                half = r % 2
                nslot = (r + NBUF - 1) % NBUF
                nr = r + NBUF - 1
                if nr < SUB:
                    stage_and_fire(g, nr, nslot)
                else:
                    @pl.when(g < sg - 1)
                    def _():
                        stage_and_fire(g + 1, nr - SUB, nslot)
                wait_gather(slot)
                if r < 2:
                    @pl.when(g > 0)
                    def _():
                        wait_write(half)
                else:
                    wait_write(half)
                transpose_write(g, r, slot, half)
            return carry

        lax.fori_loop(0, sg, body, 0)
        wait_write(0)
        wait_write(1)

    return emb


def kernel(word_ids, word_emb_table):
    batch, seq = word_ids.shape
    vocab, dim = word_emb_table.shape
    ids_lin = (
        word_ids.astype(jnp.int32)
        .T.reshape(seq // SUB, SUB, batch // LANE, LANE)
        .transpose(0, 2, 1, 3)
    )
    table_lines = word_emb_table.reshape(vocab // 2, 2 * dim)
    emb = _build(batch, seq, dim)
    out5 = emb(ids_lin, table_lines).reshape(
        seq, dim // SUB, batch // LANE, SUB, LANE
    )
    return out5.transpose(2, 4, 0, 1, 3).reshape(batch, seq, dim)


# speedup vs baseline: 2.2255x; 2.2128x over previous
import functools

import jax
import jax.numpy as jnp
from jax import lax
from jax.experimental import pallas as pl
from jax.experimental.pallas import tpu as pltpu
from jax.experimental.pallas import tpu_sc as plsc

NC = 2
NS = 16
NW = NC * NS
LANE = 128
SUB = 8
NBUF = 8


@functools.lru_cache(maxsize=None)
def _build(batch, seq, dim):
    bt = batch // LANE
    assert bt == NW and seq % SUB == 0
    sg = seq // SUB
    lg = LANE // 16

    mesh = plsc.VectorSubcoreMesh(core_axis_name="c", subcore_axis_name="s")

    @functools.partial(
        pl.kernel,
        mesh=mesh,
        out_type=jax.ShapeDtypeStruct((batch, seq * 2 * dim), jnp.float32),
        compiler_params=pltpu.CompilerParams(
            use_tc_tiling_on_sc=False, needs_layout_passes=False
        ),
        scratch_types=[
            pltpu.VMEM((sg, SUB, LANE), jnp.int32),
            pltpu.VMEM((NBUF, LANE), jnp.int32),
            pltpu.VMEM((NBUF, LANE, dim), jnp.float32),
            [pltpu.SemaphoreType.DMA] * NBUF,
            [pltpu.SemaphoreType.DMA] * NBUF,
        ],
    )
    def emb(ids_hbm, table_hbm, out_hbm, idx_v, stage, bufs, gsem, wsem):
        w = lax.axis_index("s") * NC + lax.axis_index("c")

        pltpu.sync_copy(ids_hbm.at[:, w], idx_v)

        def stage_and_fire(g, r, slot):
            for l in range(lg):
                iv = idx_v[g, r, pl.ds(16 * l, 16)]
                stage[slot, pl.ds(16 * l, 16)] = iv + iv
            pltpu.make_async_copy(
                table_hbm.at[stage.at[slot]], bufs.at[slot], gsem[slot]
            ).start()

        def wait_gather(slot):
            pltpu.make_async_copy(
                table_hbm.at[stage.at[slot]], bufs.at[slot], gsem[slot]
            ).wait()

        def out_slice(g, r):
            s = g * SUB + r
            return out_hbm.at[pl.ds(w * LANE, LANE), pl.ds(s * 2 * dim, dim)]

        def fire_write(g, r, slot):
            pltpu.make_async_copy(
                bufs.at[slot], out_slice(g, r), wsem[slot]
            ).start()

        def wait_write(g, r, slot):
            pltpu.make_async_copy(
                bufs.at[slot], out_slice(g, r), wsem[slot]
            ).wait()

        for k in range(NBUF - 1):
            stage_and_fire(0, k, k)

        def body(g, carry):
            for r in range(SUB):
                slot = r % NBUF
                nslot = (r + NBUF - 1) % NBUF
                nr = r + NBUF - 1
                if nr < SUB:
                    if r == 0:
                        @pl.when(g > 0)
                        def _():
                            wait_write(g, r, nslot)
                    else:
                        wait_write(g, r, nslot)
                    stage_and_fire(g, nr, nslot)
                else:
                    @pl.when(g < sg - 1)
                    def _():
                        wait_write(g, r, nslot)
                        stage_and_fire(g + 1, nr - SUB, nslot)
                wait_gather(slot)
                fire_write(g, r, slot)
            return carry

        lax.fori_loop(0, sg, body, 0)
        for k in range(NBUF):
            wait_write(sg - 1, k, k)

    return emb


def kernel(word_ids, word_emb_table):
    batch, seq = word_ids.shape
    vocab, dim = word_emb_table.shape
    ids_lin = (
        word_ids.astype(jnp.int32)
        .T.reshape(seq // SUB, SUB, batch // LANE, LANE)
        .transpose(0, 2, 1, 3)
    )
    table_rows = jnp.pad(word_emb_table, ((0, 0), (0, dim))).reshape(
        2 * vocab, dim
    )
    emb = _build(batch, seq, dim)
    out_pad = emb(ids_lin, table_rows).reshape(batch, seq, 2 * dim)
    return out_pad[:, :, :dim]
